# Initial kernel scaffold; baseline (speedup 1.0000x reference)
import jax
import jax.numpy as jnp
from jax.experimental import pallas as pl


def kernel(x, edge_index, batch, W1_rel, W1_root, b1, W2_rel, W2_root, b2, W3_rel, W3_root, b3):
    raise NotImplementedError("write your pallas kernel here")


# speedup vs baseline: 6.1706x; 6.1706x over previous
import functools

import jax
import jax.numpy as jnp
from jax import lax
from jax.experimental import pallas as pl
from jax.experimental.pallas import tpu as pltpu
from jax.experimental.pallas import tpu_sc as plsc

_N = 10000
_E = 320000
_D = 128
_H = 128
_C = 16
_G = 64

_NC = 2
_NS = 16
_CHUNK = 128
_NCHUNKS = _E // _CHUNK
_N_PAD = 10240
_ROWS_PER_TILE = _N_PAD // _NS


def _seg_sum_body(feat, y_hbm, src_hbm, dst_hbm, out_hbm,
                  acc_sh, zbuf, src_v, dst_v, rows_v, sem):
    c = lax.axis_index("c")
    s = lax.axis_index("s")

    z16 = jnp.zeros((16,), jnp.float32)

    def zrow(i, carry):
        for j in range(feat // 16):
            zbuf[i, pl.ds(16 * j, 16)] = z16
        return carry

    lax.fori_loop(0, 128, zrow, 0)
    base_r = s * _ROWS_PER_TILE
    for k in range(_ROWS_PER_TILE // 128):
        pltpu.sync_copy(zbuf, acc_sh.at[pl.ds(base_r + k * 128, 128)])
    plsc.subcore_barrier()

    w = c * _NS + s
    extra = _NCHUNKS - (_NCHUNKS // (_NC * _NS)) * (_NC * _NS)
    per = _NCHUNKS // (_NC * _NS)
    cnt = per + jnp.where(w < extra, 1, 0)
    start = per * w + jnp.minimum(w, extra)

    def chunk(i, carry):
        off = pl.multiple_of((start + i) * _CHUNK, 8)
        pltpu.sync_copy(src_hbm.at[pl.ds(off, _CHUNK)], src_v)
        pltpu.sync_copy(dst_hbm.at[pl.ds(off, _CHUNK)], dst_v)
        pltpu.async_copy(y_hbm.at[src_v], rows_v, sem).wait()
        pltpu.sync_copy(rows_v, acc_sh.at[dst_v], add=True)
        return carry

    lax.fori_loop(0, cnt, chunk, 0)
    plsc.subcore_barrier()

    row_off = c * _N_PAD + base_r
    pltpu.sync_copy(acc_sh.at[pl.ds(base_r, _ROWS_PER_TILE)],
                    out_hbm.at[pl.ds(row_off, _ROWS_PER_TILE)])


@functools.lru_cache(maxsize=None)
def _make_seg_sum(feat):
    mesh = plsc.VectorSubcoreMesh(core_axis_name="c", subcore_axis_name="s",
                                  num_cores=_NC, num_subcores=_NS)
    return pl.kernel(
        functools.partial(_seg_sum_body, feat),
        out_type=jax.ShapeDtypeStruct((_NC * _N_PAD, feat), jnp.float32),
        mesh=mesh,
        scratch_types=[
            pltpu.VMEM_SHARED((_N_PAD, feat), jnp.float32),
            pltpu.VMEM((128, feat), jnp.float32),
            pltpu.VMEM((_CHUNK,), jnp.int32),
            pltpu.VMEM((_CHUNK,), jnp.int32),
            pltpu.VMEM((_CHUNK, feat), jnp.float32),
            pltpu.SemaphoreType.DMA,
        ],
    )


_BLK = 1000
_GRID = _N // _BLK


def _front_body(x_ref, wrel_ref, wroot_ref, b_ref, y_ref, r_ref):
    xb = x_ref[...]
    y_ref[...] = jnp.dot(xb, wrel_ref[...], preferred_element_type=jnp.float32)
    r_ref[...] = jnp.dot(xb, wroot_ref[...],
                         preferred_element_type=jnp.float32) + b_ref[...]


def _mid_body(fo, acc_a_ref, acc_b_ref, r_ref, wrel_ref, wroot_ref, b_ref,
              y_ref, r2_ref):
    h = jnp.maximum(acc_a_ref[...] + acc_b_ref[...] + r_ref[...], 0.0)
    y_ref[...] = jnp.dot(h, wrel_ref[...], preferred_element_type=jnp.float32)
    r2_ref[...] = jnp.dot(h, wroot_ref[...],
                          preferred_element_type=jnp.float32) + b_ref[...]


def _make_front(fin, fo):
    return pl.pallas_call(
        _front_body,
        grid=(_GRID,),
        in_specs=[
            pl.BlockSpec((_BLK, fin), lambda i: (i, 0)),
            pl.BlockSpec((fin, fo), lambda i: (0, 0)),
            pl.BlockSpec((fin, fo), lambda i: (0, 0)),
            pl.BlockSpec((1, fo), lambda i: (0, 0)),
        ],
        out_specs=[
            pl.BlockSpec((_BLK, fo), lambda i: (i, 0)),
            pl.BlockSpec((_BLK, fo), lambda i: (i, 0)),
        ],
        out_shape=[
            jax.ShapeDtypeStruct((_N, fo), jnp.float32),
            jax.ShapeDtypeStruct((_N, fo), jnp.float32),
        ],
    )


def _make_mid(fin, fo):
    return pl.pallas_call(
        functools.partial(_mid_body, fo),
        grid=(_GRID,),
        in_specs=[
            pl.BlockSpec((_BLK, fin), lambda i: (i, 0)),
            pl.BlockSpec((_BLK, fin), lambda i: (i, 0)),
            pl.BlockSpec((_BLK, fin), lambda i: (i, 0)),
            pl.BlockSpec((fin, fo), lambda i: (0, 0)),
            pl.BlockSpec((fin, fo), lambda i: (0, 0)),
            pl.BlockSpec((1, fo), lambda i: (0, 0)),
        ],
        out_specs=[
            pl.BlockSpec((_BLK, fo), lambda i: (i, 0)),
            pl.BlockSpec((_BLK, fo), lambda i: (i, 0)),
        ],
        out_shape=[
            jax.ShapeDtypeStruct((_N, fo), jnp.float32),
            jax.ShapeDtypeStruct((_N, fo), jnp.float32),
        ],
    )


_front128 = _make_front(_D, _H)
_mid128 = _make_mid(_H, _H)
_mid16 = _make_mid(_H, _C)


def _pool_body(acc_a_ref, acc_b_ref, r_ref, batch_ref, out_ref, cnt_ref):
    i = pl.program_id(0)
    t = (acc_a_ref[...] + acc_b_ref[...] + r_ref[...])[:, :_C]
    bvec = batch_ref[0, 0, :]
    onehot = (bvec[:, None] ==
              lax.broadcasted_iota(jnp.int32, (_BLK, _G), 1)).astype(jnp.float32)
    dims = (((0,), (0,)), ((), ()))
    sums = lax.dot_general(onehot, t, dims, preferred_element_type=jnp.float32)
    cnts = lax.dot_general(onehot, jnp.ones((_BLK, _C), jnp.float32), dims,
                           preferred_element_type=jnp.float32)

    @pl.when(i == 0)
    def _():
        out_ref[...] = sums
        cnt_ref[...] = cnts

    @pl.when(i > 0)
    def _():
        out_ref[...] += sums
        cnt_ref[...] += cnts

    @pl.when(i == pl.num_programs(0) - 1)
    def _():
        out_ref[...] = out_ref[...] / jnp.maximum(cnt_ref[...], 1.0)


_pool = pl.pallas_call(
    _pool_body,
    grid=(_GRID,),
    in_specs=[
        pl.BlockSpec((_BLK, _H), lambda i: (i, 0)),
        pl.BlockSpec((_BLK, _H), lambda i: (i, 0)),
        pl.BlockSpec((_BLK, _H), lambda i: (i, 0)),
        pl.BlockSpec((1, 1, _BLK), lambda i: (i, 0, 0)),
    ],
    out_specs=pl.BlockSpec((_G, _C), lambda i: (0, 0)),
    out_shape=jax.ShapeDtypeStruct((_G, _C), jnp.float32),
    scratch_shapes=[pltpu.VMEM((_G, _C), jnp.float32)],
)


def kernel(x, edge_index, batch, W1_rel, W1_root, b1, W2_rel, W2_root, b2,
           W3_rel, W3_root, b3):
    src = edge_index[0]
    dst = edge_index[1]
    seg128 = _make_seg_sum(_H)
    W3_rel_p = jnp.pad(W3_rel, ((0, 0), (0, _H - _C)))
    W3_root_p = jnp.pad(W3_root, ((0, 0), (0, _H - _C)))
    b3_p = jnp.pad(b3, (0, _H - _C))
    y1, r1 = _front128(x, W1_rel, W1_root, b1.reshape(1, _H))
    acc1 = seg128(y1, src, dst)
    y2, r2 = _mid128(acc1[:_N], acc1[_N_PAD:_N_PAD + _N], r1, W2_rel, W2_root,
                     b2.reshape(1, _H))
    acc2 = seg128(y2, src, dst)
    y3, r3 = _mid128(acc2[:_N], acc2[_N_PAD:_N_PAD + _N], r2, W3_rel_p,
                     W3_root_p, b3_p.reshape(1, _H))
    acc3 = seg128(y3, src, dst)
    return _pool(acc3[:_N], acc3[_N_PAD:_N_PAD + _N], r3,
                 batch.reshape(_GRID, 1, _BLK))


# speedup vs baseline: 9.1389x; 1.4810x over previous
import functools

import jax
import jax.numpy as jnp
from jax import lax
from jax.experimental import pallas as pl
from jax.experimental.pallas import tpu as pltpu
from jax.experimental.pallas import tpu_sc as plsc

_N = 10000
_E = 320000
_D = 128
_H = 128
_C = 16
_G = 64

_NC = 2
_NS = 16
_CHUNK = 80
_CPT = _E // (_NC * _NS * _CHUNK)
_N_PAD = 10240
_ROWS_PER_TILE = _N_PAD // _NS


def _seg_sum_body(feat, y_hbm, src_hbm, dst_hbm, out_hbm,
                  acc_sh, zbuf, src_a, dst_a, rows_a, src_b, dst_b, rows_b,
                  sem_ia, sem_ib, sem_ga, sem_gb):
    c = lax.axis_index("c")
    s = lax.axis_index("s")

    z16 = jnp.zeros((16,), jnp.float32)

    def zrow(i, carry):
        for j in range(feat // 16):
            zbuf[i, pl.ds(16 * j, 16)] = z16
        return carry

    lax.fori_loop(0, 128, zrow, 0)
    base_r = s * _ROWS_PER_TILE
    for k in range(_ROWS_PER_TILE // 128):
        pltpu.sync_copy(zbuf, acc_sh.at[pl.ds(base_r + k * 128, 128)])
    plsc.subcore_barrier()

    w = c * _NS + s
    ebase = w * (_CPT * _CHUNK)

    def load_idx(j, src_v, dst_v, sem):
        off = pl.multiple_of(ebase + j * _CHUNK, 8)
        h0 = pltpu.async_copy(src_hbm.at[pl.ds(off, _CHUNK)], src_v, sem)
        h1 = pltpu.async_copy(dst_hbm.at[pl.ds(off, _CHUNK)], dst_v, sem)
        return h0, h1

    def wait_gather(src_v, rows_v, sem):
        pltpu.make_async_copy(y_hbm.at[src_v], rows_v, sem).wait()

    h0, h1 = load_idx(0, src_a, dst_a, sem_ia)
    h0.wait()
    h1.wait()
    pltpu.async_copy(y_hbm.at[src_a], rows_a, sem_ga)

    def body(k, carry):
        ja = 2 * k
        i0, i1 = load_idx(ja + 1, src_b, dst_b, sem_ib)
        wait_gather(src_a, rows_a, sem_ga)
        i0.wait()
        i1.wait()
        pltpu.async_copy(y_hbm.at[src_b], rows_b, sem_gb)
        pltpu.sync_copy(rows_a, acc_sh.at[dst_a], add=True)
        i0, i1 = load_idx(ja + 2, src_a, dst_a, sem_ia)
        wait_gather(src_b, rows_b, sem_gb)
        i0.wait()
        i1.wait()
        pltpu.async_copy(y_hbm.at[src_a], rows_a, sem_ga)
        pltpu.sync_copy(rows_b, acc_sh.at[dst_b], add=True)
        return carry

    lax.fori_loop(0, (_CPT - 1) // 2, body, 0)
    wait_gather(src_a, rows_a, sem_ga)
    pltpu.sync_copy(rows_a, acc_sh.at[dst_a], add=True)
    plsc.subcore_barrier()

    row_off = c * _N_PAD + base_r
    pltpu.sync_copy(acc_sh.at[pl.ds(base_r, _ROWS_PER_TILE)],
                    out_hbm.at[pl.ds(row_off, _ROWS_PER_TILE)])


@functools.lru_cache(maxsize=None)
def _make_seg_sum(feat):
    mesh = plsc.VectorSubcoreMesh(core_axis_name="c", subcore_axis_name="s",
                                  num_cores=_NC, num_subcores=_NS)
    return pl.kernel(
        functools.partial(_seg_sum_body, feat),
        out_type=jax.ShapeDtypeStruct((_NC * _N_PAD, feat), jnp.float32),
        mesh=mesh,
        scratch_types=[
            pltpu.VMEM_SHARED((_N_PAD, feat), jnp.float32),
            pltpu.VMEM((128, feat), jnp.float32),
            pltpu.VMEM((_CHUNK,), jnp.int32),
            pltpu.VMEM((_CHUNK,), jnp.int32),
            pltpu.VMEM((_CHUNK, feat), jnp.float32),
            pltpu.VMEM((_CHUNK,), jnp.int32),
            pltpu.VMEM((_CHUNK,), jnp.int32),
            pltpu.VMEM((_CHUNK, feat), jnp.float32),
            pltpu.SemaphoreType.DMA,
            pltpu.SemaphoreType.DMA,
            pltpu.SemaphoreType.DMA,
            pltpu.SemaphoreType.DMA,
        ],
    )


_BLK = 1000
_GRID = _N // _BLK


def _front_body(x_ref, wrel_ref, wroot_ref, b_ref, y_ref, r_ref):
    xb = x_ref[...]
    y_ref[...] = jnp.dot(xb, wrel_ref[...], preferred_element_type=jnp.float32)
    r_ref[...] = jnp.dot(xb, wroot_ref[...],
                         preferred_element_type=jnp.float32) + b_ref[...]


def _mid_body(fo, acc_a_ref, acc_b_ref, r_ref, wrel_ref, wroot_ref, b_ref,
              y_ref, r2_ref):
    h = jnp.maximum(acc_a_ref[...] + acc_b_ref[...] + r_ref[...], 0.0)
    y_ref[...] = jnp.dot(h, wrel_ref[...], preferred_element_type=jnp.float32)
    r2_ref[...] = jnp.dot(h, wroot_ref[...],
                          preferred_element_type=jnp.float32) + b_ref[...]


def _make_front(fin, fo):
    return pl.pallas_call(
        _front_body,
        grid=(_GRID,),
        in_specs=[
            pl.BlockSpec((_BLK, fin), lambda i: (i, 0)),
            pl.BlockSpec((fin, fo), lambda i: (0, 0)),
            pl.BlockSpec((fin, fo), lambda i: (0, 0)),
            pl.BlockSpec((1, fo), lambda i: (0, 0)),
        ],
        out_specs=[
            pl.BlockSpec((_BLK, fo), lambda i: (i, 0)),
            pl.BlockSpec((_BLK, fo), lambda i: (i, 0)),
        ],
        out_shape=[
            jax.ShapeDtypeStruct((_N, fo), jnp.float32),
            jax.ShapeDtypeStruct((_N, fo), jnp.float32),
        ],
    )


def _make_mid(fin, fo):
    return pl.pallas_call(
        functools.partial(_mid_body, fo),
        grid=(_GRID,),
        in_specs=[
            pl.BlockSpec((_BLK, fin), lambda i: (i, 0)),
            pl.BlockSpec((_BLK, fin), lambda i: (i, 0)),
            pl.BlockSpec((_BLK, fin), lambda i: (i, 0)),
            pl.BlockSpec((fin, fo), lambda i: (0, 0)),
            pl.BlockSpec((fin, fo), lambda i: (0, 0)),
            pl.BlockSpec((1, fo), lambda i: (0, 0)),
        ],
        out_specs=[
            pl.BlockSpec((_BLK, fo), lambda i: (i, 0)),
            pl.BlockSpec((_BLK, fo), lambda i: (i, 0)),
        ],
        out_shape=[
            jax.ShapeDtypeStruct((_N, fo), jnp.float32),
            jax.ShapeDtypeStruct((_N, fo), jnp.float32),
        ],
    )


_front128 = _make_front(_D, _H)
_mid128 = _make_mid(_H, _H)
_mid16 = _make_mid(_H, _C)


def _pool_body(acc_a_ref, acc_b_ref, r_ref, batch_ref, out_ref, cnt_ref):
    i = pl.program_id(0)
    t = (acc_a_ref[...] + acc_b_ref[...] + r_ref[...])[:, :_C]
    bvec = batch_ref[0, 0, :]
    onehot = (bvec[:, None] ==
              lax.broadcasted_iota(jnp.int32, (_BLK, _G), 1)).astype(jnp.float32)
    dims = (((0,), (0,)), ((), ()))
    sums = lax.dot_general(onehot, t, dims, preferred_element_type=jnp.float32)
    cnts = lax.dot_general(onehot, jnp.ones((_BLK, _C), jnp.float32), dims,
                           preferred_element_type=jnp.float32)

    @pl.when(i == 0)
    def _():
        out_ref[...] = sums
        cnt_ref[...] = cnts

    @pl.when(i > 0)
    def _():
        out_ref[...] += sums
        cnt_ref[...] += cnts

    @pl.when(i == pl.num_programs(0) - 1)
    def _():
        out_ref[...] = out_ref[...] / jnp.maximum(cnt_ref[...], 1.0)


_pool = pl.pallas_call(
    _pool_body,
    grid=(_GRID,),
    in_specs=[
        pl.BlockSpec((_BLK, _H), lambda i: (i, 0)),
        pl.BlockSpec((_BLK, _H), lambda i: (i, 0)),
        pl.BlockSpec((_BLK, _H), lambda i: (i, 0)),
        pl.BlockSpec((1, 1, _BLK), lambda i: (i, 0, 0)),
    ],
    out_specs=pl.BlockSpec((_G, _C), lambda i: (0, 0)),
    out_shape=jax.ShapeDtypeStruct((_G, _C), jnp.float32),
    scratch_shapes=[pltpu.VMEM((_G, _C), jnp.float32)],
)


def kernel(x, edge_index, batch, W1_rel, W1_root, b1, W2_rel, W2_root, b2,
           W3_rel, W3_root, b3):
    src = edge_index[0]
    dst = edge_index[1]
    seg128 = _make_seg_sum(_H)
    W3_rel_p = jnp.pad(W3_rel, ((0, 0), (0, _H - _C)))
    W3_root_p = jnp.pad(W3_root, ((0, 0), (0, _H - _C)))
    b3_p = jnp.pad(b3, (0, _H - _C))
    y1, r1 = _front128(x, W1_rel, W1_root, b1.reshape(1, _H))
    acc1 = seg128(y1, src, dst)
    y2, r2 = _mid128(acc1[:_N], acc1[_N_PAD:_N_PAD + _N], r1, W2_rel, W2_root,
                     b2.reshape(1, _H))
    acc2 = seg128(y2, src, dst)
    y3, r3 = _mid128(acc2[:_N], acc2[_N_PAD:_N_PAD + _N], r2, W3_rel_p,
                     W3_root_p, b3_p.reshape(1, _H))
    acc3 = seg128(y3, src, dst)
    return _pool(acc3[:_N], acc3[_N_PAD:_N_PAD + _N], r3,
                 batch.reshape(_GRID, 1, _BLK))
